# baseline (speedup 1.0000x reference)
import jax
import jax.numpy as jnp
from jax import lax
from jax.experimental import pallas as pl
from jax.experimental.pallas import tpu as pltpu
from jax.experimental.pallas import tpu_sc as plsc

N = 10000
E = 320000
G = 128
IN_DIM = 128
OUT_DIM = 10

NPAD = 10240
RPT = NPAD // 16
CH = 128
NROW = 2624
E_PAD = NROW * CH


def _sc_edge_pass(do):
    hw = do // 2
    grp = hw // 16
    ch = 64 if hw == 64 else 128
    cpw = E_PAD // (16 * ch)
    nbuf, gdepth = 4, 2
    mesh = plsc.VectorSubcoreMesh(core_axis_name="c", subcore_axis_name="s")

    def body(ha_hbm, hb_hbm, ss_hbm, sd_hbm, src_hbm, dst_hbm,
             out_hbm, den_hbm,
             srcv, dstv, ss_v, sd_v, acc_sh, den_sh, *bufflat):
        cid = lax.axis_index("c")
        sid = lax.axis_index("s")
        r0 = sid * RPT
        rows = bufflat[0:nbuf]
        exs = bufflat[nbuf:2 * nbuf]
        sg = bufflat[2 * nbuf:3 * nbuf]
        sr = bufflat[3 * nbuf:4 * nbuf]
        se = bufflat[4 * nbuf:5 * nbuf]
        rows0, ex0 = rows[0], exs[0]

        stg = [pltpu.async_copy(ss_hbm, ss_v, sg[0]),
               pltpu.async_copy(sd_hbm, sd_v, sg[1]),
               pltpu.async_copy(src_hbm.at[pl.ds(sid * cpw, cpw)], srcv,
                                sg[2]),
               pltpu.async_copy(dst_hbm.at[pl.ds(sid * cpw, cpw)], dstv,
                                sg[3])]

        zero16 = jnp.zeros((16,), jnp.float32)

        def zrow(i, _):
            for c in range(grp):
                rows0[i, pl.ds(c * 16, 16)] = zero16
            return 0

        lax.fori_loop(0, ch, zrow, 0)
        for i in range(ch // 16):
            ex0[pl.ds(i * 16, 16)] = zero16
        zacc = [pltpu.async_copy(rows0, acc_sh.at[pl.ds(r0 + j * ch, ch)],
                                 sr[0]) for j in range(RPT // ch)]
        zden = [pltpu.async_copy(ex0, den_sh.at[pl.ds(r0 + j * ch, ch)],
                                 sr[1]) for j in range(RPT // ch)]
        for d in stg + zacc + zden:
            d.wait()

        plsc.subcore_barrier()

        def issue_gather(j, b):
            @pl.when(cid == 0)
            def _():
                pltpu.async_copy(ha_hbm.at[srcv.at[j]], rows[b], sg[b])

            @pl.when(cid == 1)
            def _():
                pltpu.async_copy(hb_hbm.at[srcv.at[j]], rows[b], sg[b])

        def wait_gather(b):
            pltpu.make_async_copy(ha_hbm.at[srcv.at[0]], rows[b], sg[b]).wait()

        def issue_scatter(j, b):
            pltpu.async_copy(rows[b], acc_sh.at[dstv.at[j]], sr[b], add=True)

            @pl.when(b % 2 == cid)
            def _():
                pltpu.async_copy(exs[b], den_sh.at[dstv.at[j]], se[b],
                                 add=True)

        def wait_scatter(b):
            pltpu.make_async_copy(rows[b], acc_sh.at[dstv.at[0]],
                                  sr[b]).wait()

            @pl.when(b % 2 == cid)
            def _():
                pltpu.make_async_copy(exs[b], den_sh.at[dstv.at[0]],
                                      se[b]).wait()

        for q in range(gdepth):
            issue_gather(q, q)

        def group(jo, _):
            for b in range(nbuf):
                j = jo * nbuf + b
                rows_v, ex_v = rows[b], exs[b]
                bg = (b + gdepth) % nbuf

                @pl.when(j + gdepth < cpw)
                def _():
                    @pl.when(j >= nbuf - gdepth)
                    def _():
                        wait_scatter(bg)
                    issue_gather(j + gdepth, bg)

                wait_gather(b)

                def proc(g, _):
                    s16 = srcv[j, pl.ds(g * 16, 16)]
                    d16 = dstv[j, pl.ds(g * 16, 16)]
                    e = (plsc.load_gather(ss_v, [s16])
                         + plsc.load_gather(sd_v, [d16]))
                    ex16 = jnp.exp(jnp.maximum(e, 0.2 * e))
                    ex_v[pl.ds(g * 16, 16)] = ex16
                    for l in range(16):
                        k = g * 16 + l
                        exk = ex16[l]
                        for c in range(grp):
                            sl = pl.ds(c * 16, 16)
                            rows_v[k, sl] = rows_v[k, sl] * exk
                    return 0

                lax.fori_loop(0, ch // 16, proc, 0)
                issue_scatter(j, b)
            return 0

        lax.fori_loop(0, cpw // nbuf, group, 0)
        for b in range(nbuf):
            wait_scatter(b)
        plsc.subcore_barrier()

        out1 = pltpu.async_copy(acc_sh.at[pl.ds(r0, RPT)],
                                out_hbm.at[cid, pl.ds(r0, RPT)], sg[0])
        out2 = pltpu.async_copy(den_sh.at[pl.ds(r0, RPT)],
                                den_hbm.at[cid, pl.ds(r0, RPT)], sg[1])
        out1.wait()
        out2.wait()

    return pl.kernel(
        body,
        out_type=(jax.ShapeDtypeStruct((2, NPAD, hw), jnp.float32),
                  jax.ShapeDtypeStruct((2, NPAD), jnp.float32)),
        mesh=mesh,
        compiler_params=pltpu.CompilerParams(needs_layout_passes=False,
                                             use_tc_tiling_on_sc=False),
        scratch_types=(
            [pltpu.VMEM((cpw, ch), jnp.int32),
             pltpu.VMEM((cpw, ch), jnp.int32),
             pltpu.VMEM((NPAD,), jnp.float32),
             pltpu.VMEM((NPAD,), jnp.float32),
             pltpu.VMEM_SHARED((NPAD, hw), jnp.float32),
             pltpu.VMEM_SHARED((NPAD,), jnp.float32)]
            + [pltpu.VMEM((ch, hw), jnp.float32)] * nbuf
            + [pltpu.VMEM((ch,), jnp.float32)] * nbuf
            + [pltpu.SemaphoreType.DMA] * (3 * nbuf)),
    )


def _split(hn, hn_ref):
    hw = hn.shape[1] // 2
    hn_ref[0] = hn[:, :hw]
    hn_ref[1] = hn[:, hw:]


def _k0_body(x_ref, w_ref, as_ref, ad_ref, h_ref, ss_ref, sd_ref):
    h = jnp.dot(x_ref[:], w_ref[:], preferred_element_type=jnp.float32)
    _split(h, h_ref)
    ss_ref[:] = jnp.dot(h, as_ref[:], preferred_element_type=jnp.float32)
    sd_ref[:] = jnp.dot(h, ad_ref[:], preferred_element_type=jnp.float32)


def _combine_bn(outp, denp, b, bn_g, bn_b):
    den = (denp[0] + denp[1] + 1e-16)[:, None]
    num = jnp.concatenate([outp[0], outp[1]], axis=1)
    do = num.shape[1]
    mask = lax.broadcasted_iota(jnp.int32, (NPAD, do), 0) < N
    g = num / den + b
    g = jnp.maximum(g, 0.01 * g)
    g = jnp.where(mask, g, 0.0)
    mu = jnp.sum(g, axis=0, keepdims=True) / N
    gc = jnp.where(mask, g - mu, 0.0)
    var = jnp.sum(gc * gc, axis=0, keepdims=True) / N
    hbn = bn_g * gc * jax.lax.rsqrt(var + 1e-5) + bn_b
    return jnp.where(mask, hbn, 0.0)


def _ep_body(outp_ref, denp_ref, b_ref, g_ref, beta_ref,
             wn_ref, asn_ref, adn_ref, hn_ref, ssn_ref, sdn_ref):
    hbn = _combine_bn(outp_ref[:], denp_ref[:],
                      b_ref[:], g_ref[:], beta_ref[:])
    hn = jnp.dot(hbn, wn_ref[:], preferred_element_type=jnp.float32)
    _split(hn, hn_ref)
    ssn_ref[:] = jnp.dot(hn, asn_ref[:], preferred_element_type=jnp.float32)
    sdn_ref[:] = jnp.dot(hn, adn_ref[:], preferred_element_type=jnp.float32)


def _final_body(outp_ref, denp_ref, b_ref, g_ref,
                beta_ref, batch_ref, fw_ref, fb_ref, cw_ref, cb_ref, out_ref):
    hbn = _combine_bn(outp_ref[:], denp_ref[:],
                      b_ref[:], g_ref[:], beta_ref[:])
    gi = lax.broadcasted_iota(jnp.int32, (NPAD, G), 1)
    p = (batch_ref[:] == gi).astype(jnp.float32)
    pooled = lax.dot_general(p, hbn, (((0,), (0,)), ((), ())),
                             preferred_element_type=jnp.float32)
    t = jnp.dot(pooled, fw_ref[:], preferred_element_type=jnp.float32) + fb_ref[:]
    t = jnp.maximum(t, 0.01 * t)
    o = jnp.dot(t, cw_ref[:], preferred_element_type=jnp.float32) + cb_ref[:]
    out_ref[:] = 1.0 / (1.0 + jnp.exp(-o))


def _tc_call(body, out_shapes):
    return pl.pallas_call(
        body,
        out_shape=[jax.ShapeDtypeStruct(s, jnp.float32) for s in out_shapes])


def kernel(x, edge_index, batch,
           W1, a_src1, a_dst1, b1, bn1_g, bn1_b,
           W2, a_src2, a_dst2, b2, bn2_g, bn2_b,
           W3, a_src3, a_dst3, b3, bn3_g, bn3_b,
           W4, a_src4, a_dst4, b4, bn4_g, bn4_b,
           fc1_W, fc1_b, cls_W, cls_b):
    xp = jnp.pad(x, ((0, NPAD - N), (0, 0)))
    loops = jnp.arange(N, dtype=edge_index.dtype)
    srcp = jnp.pad(jnp.concatenate([edge_index[0], loops]), (0, E_PAD - E - N),
                   constant_values=N)
    dstp = jnp.pad(jnp.concatenate([edge_index[1], loops]), (0, E_PAD - E - N),
                   constant_values=N)
    batchp = jnp.pad(batch, (0, NPAD - N), constant_values=G)[:, None]

    col = lambda v: v.reshape(-1, 1)
    row = lambda v: v.reshape(1, -1)

    dims = [32, 64, 128, 64]
    As = [a_src1, a_src2, a_src3, a_src4]
    Ad = [a_dst1, a_dst2, a_dst3, a_dst4]
    Ws = [W1, W2, W3, W4]
    Bs = [b1, b2, b3, b4]
    Gs = [bn1_g, bn2_g, bn3_g, bn4_g]
    Bt = [bn1_b, bn2_b, bn3_b, bn4_b]

    hs, ss, sd = _tc_call(_k0_body, [(2, NPAD, 16), (NPAD, 1), (NPAD, 1)])(
        xp, W1, col(a_src1), col(a_dst1))

    for i in range(4):
        do = dims[i]
        ch = 64 if do == 128 else 128
        outp, denp = _sc_edge_pass(do)(
            hs[0], hs[1], ss.reshape(-1), sd.reshape(-1),
            srcp.reshape(E_PAD // ch, ch), dstp.reshape(E_PAD // ch, ch))
        if i < 3:
            dn = dims[i + 1]
            hs, ss, sd = _tc_call(
                _ep_body, [(2, NPAD, dn // 2), (NPAD, 1), (NPAD, 1)])(
                    outp, denp, row(Bs[i]), row(Gs[i]), row(Bt[i]),
                    Ws[i + 1], col(As[i + 1]), col(Ad[i + 1]))
        else:
            (out,) = _tc_call(_final_body, [(G, OUT_DIM)])(
                outp, denp, row(Bs[i]), row(Gs[i]), row(Bt[i]),
                batchp, fc1_W, row(fc1_b), cls_W, row(cls_b))
    return out


# speedup vs baseline: 1.0983x; 1.0983x over previous
import jax
import jax.numpy as jnp
from jax import lax
from jax.experimental import pallas as pl
from jax.experimental.pallas import tpu as pltpu
from jax.experimental.pallas import tpu_sc as plsc

N = 10000
E = 320000
G = 128
IN_DIM = 128
OUT_DIM = 10

NPAD = 10240
RPT = NPAD // 16
CH = 128
NROW = 2624
E_PAD = NROW * CH


def _sc_edge_pass(do):
    hw = do // 2
    grp = hw // 16
    ch = 128
    if hw == 64:
        nbuf, gdepth, cpw = 3, 1, 162
    else:
        nbuf, gdepth, cpw = 4, 2, 164
    mesh = plsc.VectorSubcoreMesh(core_axis_name="c", subcore_axis_name="s")

    def body(ha_hbm, hb_hbm, ss_hbm, sd_hbm, src_hbm, dst_hbm,
             out_hbm, den_hbm,
             srcv, dstv, ss_v, sd_v, acc_sh, den_sh, *bufflat):
        cid = lax.axis_index("c")
        sid = lax.axis_index("s")
        r0 = sid * RPT
        rows = bufflat[0:nbuf]
        exs = bufflat[nbuf:2 * nbuf]
        sg = bufflat[2 * nbuf:3 * nbuf]
        sr = bufflat[3 * nbuf:4 * nbuf]
        se = bufflat[4 * nbuf:5 * nbuf]
        rows0, ex0 = rows[0], exs[0]

        stg = [pltpu.async_copy(ss_hbm, ss_v, sg[0]),
               pltpu.async_copy(sd_hbm, sd_v, sg[1]),
               pltpu.async_copy(src_hbm.at[pl.ds(sid * cpw, cpw)], srcv,
                                sg[2]),
               pltpu.async_copy(dst_hbm.at[pl.ds(sid * cpw, cpw)], dstv,
                                se[0])]

        zero16 = jnp.zeros((16,), jnp.float32)

        def zrow(i, _):
            for c in range(grp):
                rows0[i, pl.ds(c * 16, 16)] = zero16
            return 0

        lax.fori_loop(0, ch, zrow, 0)
        for i in range(ch // 16):
            ex0[pl.ds(i * 16, 16)] = zero16
        zacc = [pltpu.async_copy(rows0, acc_sh.at[pl.ds(r0 + j * ch, ch)],
                                 sr[0]) for j in range(RPT // ch)]
        zden = [pltpu.async_copy(ex0, den_sh.at[pl.ds(r0 + j * ch, ch)],
                                 sr[1]) for j in range(RPT // ch)]
        for d in stg + zacc + zden:
            d.wait()

        plsc.subcore_barrier()

        def issue_gather(j, b):
            @pl.when(cid == 0)
            def _():
                pltpu.async_copy(ha_hbm.at[srcv.at[j]], rows[b], sg[b])

            @pl.when(cid == 1)
            def _():
                pltpu.async_copy(hb_hbm.at[srcv.at[j]], rows[b], sg[b])

        def wait_gather(b):
            pltpu.make_async_copy(ha_hbm.at[srcv.at[0]], rows[b], sg[b]).wait()

        def issue_scatter(j, b):
            pltpu.async_copy(rows[b], acc_sh.at[dstv.at[j]], sr[b], add=True)

            @pl.when(b % 2 == cid)
            def _():
                pltpu.async_copy(exs[b], den_sh.at[dstv.at[j]], se[b],
                                 add=True)

        def wait_scatter(b):
            pltpu.make_async_copy(rows[b], acc_sh.at[dstv.at[0]],
                                  sr[b]).wait()

            @pl.when(b % 2 == cid)
            def _():
                pltpu.make_async_copy(exs[b], den_sh.at[dstv.at[0]],
                                      se[b]).wait()

        for q in range(gdepth):
            issue_gather(q, q)

        def group(jo, _):
            for b in range(nbuf):
                j = jo * nbuf + b
                rows_v, ex_v = rows[b], exs[b]
                bg = (b + gdepth) % nbuf

                @pl.when(j + gdepth < cpw)
                def _():
                    @pl.when(j >= nbuf - gdepth)
                    def _():
                        wait_scatter(bg)
                    issue_gather(j + gdepth, bg)

                wait_gather(b)

                def proc(g, _):
                    s16 = srcv[j, pl.ds(g * 16, 16)]
                    d16 = dstv[j, pl.ds(g * 16, 16)]
                    e = (plsc.load_gather(ss_v, [s16])
                         + plsc.load_gather(sd_v, [d16]))
                    ex16 = jnp.exp(jnp.maximum(e, 0.2 * e))
                    ex_v[pl.ds(g * 16, 16)] = ex16
                    for l in range(16):
                        k = g * 16 + l
                        exk = ex16[l]
                        for c in range(grp):
                            sl = pl.ds(c * 16, 16)
                            rows_v[k, sl] = rows_v[k, sl] * exk
                    return 0

                lax.fori_loop(0, ch // 16, proc, 0)
                issue_scatter(j, b)
            return 0

        lax.fori_loop(0, cpw // nbuf, group, 0)
        for b in range(nbuf):
            wait_scatter(b)
        plsc.subcore_barrier()

        out1 = pltpu.async_copy(acc_sh.at[pl.ds(r0, RPT)],
                                out_hbm.at[cid, pl.ds(r0, RPT)], sg[0])
        out2 = pltpu.async_copy(den_sh.at[pl.ds(r0, RPT)],
                                den_hbm.at[cid, pl.ds(r0, RPT)], sg[1])
        out1.wait()
        out2.wait()

    return pl.kernel(
        body,
        out_type=(jax.ShapeDtypeStruct((2, NPAD, hw), jnp.float32),
                  jax.ShapeDtypeStruct((2, NPAD), jnp.float32)),
        mesh=mesh,
        compiler_params=pltpu.CompilerParams(needs_layout_passes=False,
                                             use_tc_tiling_on_sc=False),
        scratch_types=(
            [pltpu.VMEM((cpw, ch), jnp.int32),
             pltpu.VMEM((cpw, ch), jnp.int32),
             pltpu.VMEM((NPAD,), jnp.float32),
             pltpu.VMEM((NPAD,), jnp.float32),
             pltpu.VMEM_SHARED((NPAD, hw), jnp.float32),
             pltpu.VMEM_SHARED((NPAD,), jnp.float32)]
            + [pltpu.VMEM((ch, hw), jnp.float32)] * nbuf
            + [pltpu.VMEM((ch,), jnp.float32)] * nbuf
            + [pltpu.SemaphoreType.DMA] * (3 * nbuf)),
    )


def _split(hn, hn_ref):
    hw = hn.shape[1] // 2
    hn_ref[0] = hn[:, :hw]
    hn_ref[1] = hn[:, hw:]


def _k0_body(x_ref, w_ref, as_ref, ad_ref, h_ref, ss_ref, sd_ref):
    h = jnp.dot(x_ref[:], w_ref[:], preferred_element_type=jnp.float32)
    _split(h, h_ref)
    ss_ref[:] = jnp.dot(h, as_ref[:], preferred_element_type=jnp.float32)
    sd_ref[:] = jnp.dot(h, ad_ref[:], preferred_element_type=jnp.float32)


def _combine_bn(outp, denp, b, bn_g, bn_b):
    den = (denp[0] + denp[1] + 1e-16)[:, None]
    num = jnp.concatenate([outp[0], outp[1]], axis=1)
    do = num.shape[1]
    mask = lax.broadcasted_iota(jnp.int32, (NPAD, do), 0) < N
    g = num / den + b
    g = jnp.maximum(g, 0.01 * g)
    g = jnp.where(mask, g, 0.0)
    mu = jnp.sum(g, axis=0, keepdims=True) / N
    gc = jnp.where(mask, g - mu, 0.0)
    var = jnp.sum(gc * gc, axis=0, keepdims=True) / N
    hbn = bn_g * gc * jax.lax.rsqrt(var + 1e-5) + bn_b
    return jnp.where(mask, hbn, 0.0)


def _ep_body(outp_ref, denp_ref, b_ref, g_ref, beta_ref,
             wn_ref, asn_ref, adn_ref, hn_ref, ssn_ref, sdn_ref):
    hbn = _combine_bn(outp_ref[:], denp_ref[:],
                      b_ref[:], g_ref[:], beta_ref[:])
    hn = jnp.dot(hbn, wn_ref[:], preferred_element_type=jnp.float32)
    _split(hn, hn_ref)
    ssn_ref[:] = jnp.dot(hn, asn_ref[:], preferred_element_type=jnp.float32)
    sdn_ref[:] = jnp.dot(hn, adn_ref[:], preferred_element_type=jnp.float32)


def _final_body(outp_ref, denp_ref, b_ref, g_ref,
                beta_ref, batch_ref, fw_ref, fb_ref, cw_ref, cb_ref, out_ref):
    hbn = _combine_bn(outp_ref[:], denp_ref[:],
                      b_ref[:], g_ref[:], beta_ref[:])
    gi = lax.broadcasted_iota(jnp.int32, (NPAD, G), 1)
    p = (batch_ref[:] == gi).astype(jnp.float32)
    pooled = lax.dot_general(p, hbn, (((0,), (0,)), ((), ())),
                             preferred_element_type=jnp.float32)
    t = jnp.dot(pooled, fw_ref[:], preferred_element_type=jnp.float32) + fb_ref[:]
    t = jnp.maximum(t, 0.01 * t)
    o = jnp.dot(t, cw_ref[:], preferred_element_type=jnp.float32) + cb_ref[:]
    out_ref[:] = 1.0 / (1.0 + jnp.exp(-o))


def _tc_call(body, out_shapes):
    return pl.pallas_call(
        body,
        out_shape=[jax.ShapeDtypeStruct(s, jnp.float32) for s in out_shapes])


def kernel(x, edge_index, batch,
           W1, a_src1, a_dst1, b1, bn1_g, bn1_b,
           W2, a_src2, a_dst2, b2, bn2_g, bn2_b,
           W3, a_src3, a_dst3, b3, bn3_g, bn3_b,
           W4, a_src4, a_dst4, b4, bn4_g, bn4_b,
           fc1_W, fc1_b, cls_W, cls_b):
    xp = jnp.pad(x, ((0, NPAD - N), (0, 0)))
    loops = jnp.arange(N, dtype=edge_index.dtype)
    srcp = jnp.pad(jnp.concatenate([edge_index[0], loops]), (0, E_PAD - E - N),
                   constant_values=N)
    dstp = jnp.pad(jnp.concatenate([edge_index[1], loops]), (0, E_PAD - E - N),
                   constant_values=N)
    batchp = jnp.pad(batch, (0, NPAD - N), constant_values=G)[:, None]

    col = lambda v: v.reshape(-1, 1)
    row = lambda v: v.reshape(1, -1)

    dims = [32, 64, 128, 64]
    As = [a_src1, a_src2, a_src3, a_src4]
    Ad = [a_dst1, a_dst2, a_dst3, a_dst4]
    Ws = [W1, W2, W3, W4]
    Bs = [b1, b2, b3, b4]
    Gs = [bn1_g, bn2_g, bn3_g, bn4_g]
    Bt = [bn1_b, bn2_b, bn3_b, bn4_b]

    hs, ss, sd = _tc_call(_k0_body, [(2, NPAD, 16), (NPAD, 1), (NPAD, 1)])(
        xp, W1, col(a_src1), col(a_dst1))

    for i in range(4):
        do = dims[i]
        ch = 128
        outp, denp = _sc_edge_pass(do)(
            hs[0], hs[1], ss.reshape(-1), sd.reshape(-1),
            srcp.reshape(E_PAD // ch, ch), dstp.reshape(E_PAD // ch, ch))
        if i < 3:
            dn = dims[i + 1]
            hs, ss, sd = _tc_call(
                _ep_body, [(2, NPAD, dn // 2), (NPAD, 1), (NPAD, 1)])(
                    outp, denp, row(Bs[i]), row(Gs[i]), row(Bt[i]),
                    Ws[i + 1], col(As[i + 1]), col(Ad[i + 1]))
        else:
            (out,) = _tc_call(_final_body, [(G, OUT_DIM)])(
                outp, denp, row(Bs[i]), row(Gs[i]), row(Bt[i]),
                batchp, fc1_W, row(fc1_b), cls_W, row(cls_b))
    return out


# speedup vs baseline: 1.3138x; 1.1963x over previous
import jax
import jax.numpy as jnp
from jax import lax
from jax.experimental import pallas as pl
from jax.experimental.pallas import tpu as pltpu
from jax.experimental.pallas import tpu_sc as plsc

N = 10000
E = 320000
G = 128
IN_DIM = 128
OUT_DIM = 10

NPAD = 10240
RPT = NPAD // 16
CH = 128
NROW = 2624
E_PAD = NROW * CH


def _sc_edge_pass(do):
    hw = do // 2
    grp = hw // 16
    ch = 128
    if hw == 64:
        nbuf, gdepth, cpw = 3, 1, 162
    else:
        nbuf, gdepth, cpw = 6, 3, 162
    mesh = plsc.VectorSubcoreMesh(core_axis_name="c", subcore_axis_name="s")

    def body(ha_hbm, hb_hbm, ss_hbm, sd_hbm, src_hbm, dst_hbm,
             out_hbm, den_hbm,
             srcv, dstv, ss_v, sd_v, acc_sh, den_sh, *bufflat):
        cid = lax.axis_index("c")
        sid = lax.axis_index("s")
        r0 = sid * RPT
        rows = bufflat[0:nbuf]
        exs = bufflat[nbuf:2 * nbuf]
        sg = bufflat[2 * nbuf:3 * nbuf]
        sr = bufflat[3 * nbuf:4 * nbuf]
        se = bufflat[4 * nbuf:5 * nbuf]
        rows0, ex0 = rows[0], exs[0]

        stg = [pltpu.async_copy(ss_hbm, ss_v, sg[0]),
               pltpu.async_copy(sd_hbm, sd_v, sg[1]),
               pltpu.async_copy(src_hbm.at[pl.ds(sid * cpw, cpw)], srcv,
                                sg[2]),
               pltpu.async_copy(dst_hbm.at[pl.ds(sid * cpw, cpw)], dstv,
                                se[0])]

        zero16 = jnp.zeros((16,), jnp.float32)

        def zrow(i, _):
            for c in range(grp):
                rows0[i, pl.ds(c * 16, 16)] = zero16
            return 0

        lax.fori_loop(0, ch, zrow, 0)
        for i in range(ch // 16):
            ex0[pl.ds(i * 16, 16)] = zero16
        zacc = [pltpu.async_copy(rows0, acc_sh.at[pl.ds(r0 + j * ch, ch)],
                                 sr[0]) for j in range(RPT // ch)]
        zden = [pltpu.async_copy(ex0, den_sh.at[pl.ds(r0 + j * ch, ch)],
                                 sr[1]) for j in range(RPT // ch)]
        for d in stg + zacc + zden:
            d.wait()

        plsc.subcore_barrier()

        def issue_gather(j, b):
            @pl.when(cid == 0)
            def _():
                pltpu.async_copy(ha_hbm.at[srcv.at[j]], rows[b], sg[b])

            @pl.when(cid == 1)
            def _():
                pltpu.async_copy(hb_hbm.at[srcv.at[j]], rows[b], sg[b])

        def wait_gather(b):
            pltpu.make_async_copy(ha_hbm.at[srcv.at[0]], rows[b], sg[b]).wait()

        def issue_scatter(j, b):
            pltpu.async_copy(rows[b], acc_sh.at[dstv.at[j]], sr[b], add=True)

            @pl.when(b % 2 == cid)
            def _():
                pltpu.async_copy(exs[b], den_sh.at[dstv.at[j]], se[b],
                                 add=True)

        def wait_scatter(b):
            pltpu.make_async_copy(rows[b], acc_sh.at[dstv.at[0]],
                                  sr[b]).wait()

            @pl.when(b % 2 == cid)
            def _():
                pltpu.make_async_copy(exs[b], den_sh.at[dstv.at[0]],
                                      se[b]).wait()

        for q in range(gdepth):
            issue_gather(q, q)

        def group(jo, _):
            for b in range(nbuf):
                j = jo * nbuf + b
                rows_v, ex_v = rows[b], exs[b]
                bg = (b + gdepth) % nbuf

                @pl.when(j + gdepth < cpw)
                def _():
                    @pl.when(j >= nbuf - gdepth)
                    def _():
                        wait_scatter(bg)
                    issue_gather(j + gdepth, bg)

                wait_gather(b)

                def proc(g, _):
                    s16 = srcv[j, pl.ds(g * 16, 16)]
                    d16 = dstv[j, pl.ds(g * 16, 16)]
                    e = (plsc.load_gather(ss_v, [s16])
                         + plsc.load_gather(sd_v, [d16]))
                    ex16 = jnp.exp(jnp.maximum(e, 0.2 * e))
                    ex_v[pl.ds(g * 16, 16)] = ex16
                    for l in range(16):
                        k = g * 16 + l
                        exk = ex16[l]
                        for c in range(grp):
                            sl = pl.ds(c * 16, 16)
                            rows_v[k, sl] = rows_v[k, sl] * exk
                    return 0

                lax.fori_loop(0, ch // 16, proc, 0)
                issue_scatter(j, b)
            return 0

        lax.fori_loop(0, cpw // nbuf, group, 0)
        for b in range(nbuf):
            wait_scatter(b)
        plsc.subcore_barrier()

        out1 = pltpu.async_copy(acc_sh.at[pl.ds(r0, RPT)],
                                out_hbm.at[cid, pl.ds(r0, RPT)], sg[0])
        out2 = pltpu.async_copy(den_sh.at[pl.ds(r0, RPT)],
                                den_hbm.at[cid, pl.ds(r0, RPT)], sg[1])
        out1.wait()
        out2.wait()

    return pl.kernel(
        body,
        out_type=(jax.ShapeDtypeStruct((2, NPAD, hw), jnp.float32),
                  jax.ShapeDtypeStruct((2, NPAD), jnp.float32)),
        mesh=mesh,
        compiler_params=pltpu.CompilerParams(needs_layout_passes=False,
                                             use_tc_tiling_on_sc=False),
        scratch_types=(
            [pltpu.VMEM((cpw, ch), jnp.int32),
             pltpu.VMEM((cpw, ch), jnp.int32),
             pltpu.VMEM((NPAD,), jnp.float32),
             pltpu.VMEM((NPAD,), jnp.float32),
             pltpu.VMEM_SHARED((NPAD, hw), jnp.float32),
             pltpu.VMEM_SHARED((NPAD,), jnp.float32)]
            + [pltpu.VMEM((ch, hw), jnp.float32)] * nbuf
            + [pltpu.VMEM((ch,), jnp.float32)] * nbuf
            + [pltpu.SemaphoreType.DMA] * (3 * nbuf)),
    )


def _split(hn, hn_ref):
    hw = hn.shape[1] // 2
    hn_ref[0] = hn[:, :hw]
    hn_ref[1] = hn[:, hw:]


def _k0_body(x_ref, w_ref, as_ref, ad_ref, h_ref, ss_ref, sd_ref):
    h = jnp.dot(x_ref[:], w_ref[:], preferred_element_type=jnp.float32)
    _split(h, h_ref)
    ss_ref[:] = jnp.dot(h, as_ref[:], preferred_element_type=jnp.float32)
    sd_ref[:] = jnp.dot(h, ad_ref[:], preferred_element_type=jnp.float32)


def _combine_bn(outp, denp, b, bn_g, bn_b):
    den = (denp[0] + denp[1] + 1e-16)[:, None]
    num = jnp.concatenate([outp[0], outp[1]], axis=1)
    do = num.shape[1]
    mask = lax.broadcasted_iota(jnp.int32, (NPAD, do), 0) < N
    g = num / den + b
    g = jnp.maximum(g, 0.01 * g)
    g = jnp.where(mask, g, 0.0)
    mu = jnp.sum(g, axis=0, keepdims=True) / N
    gc = jnp.where(mask, g - mu, 0.0)
    var = jnp.sum(gc * gc, axis=0, keepdims=True) / N
    hbn = bn_g * gc * jax.lax.rsqrt(var + 1e-5) + bn_b
    return jnp.where(mask, hbn, 0.0)


def _ep_body(outp_ref, denp_ref, b_ref, g_ref, beta_ref,
             wn_ref, asn_ref, adn_ref, hn_ref, ssn_ref, sdn_ref):
    hbn = _combine_bn(outp_ref[:], denp_ref[:],
                      b_ref[:], g_ref[:], beta_ref[:])
    hn = jnp.dot(hbn, wn_ref[:], preferred_element_type=jnp.float32)
    _split(hn, hn_ref)
    ssn_ref[:] = jnp.dot(hn, asn_ref[:], preferred_element_type=jnp.float32)
    sdn_ref[:] = jnp.dot(hn, adn_ref[:], preferred_element_type=jnp.float32)


def _final_body(outp_ref, denp_ref, b_ref, g_ref,
                beta_ref, batch_ref, fw_ref, fb_ref, cw_ref, cb_ref, out_ref):
    hbn = _combine_bn(outp_ref[:], denp_ref[:],
                      b_ref[:], g_ref[:], beta_ref[:])
    gi = lax.broadcasted_iota(jnp.int32, (NPAD, G), 1)
    p = (batch_ref[:] == gi).astype(jnp.float32)
    pooled = lax.dot_general(p, hbn, (((0,), (0,)), ((), ())),
                             preferred_element_type=jnp.float32)
    t = jnp.dot(pooled, fw_ref[:], preferred_element_type=jnp.float32) + fb_ref[:]
    t = jnp.maximum(t, 0.01 * t)
    o = jnp.dot(t, cw_ref[:], preferred_element_type=jnp.float32) + cb_ref[:]
    out_ref[:] = 1.0 / (1.0 + jnp.exp(-o))


def _tc_call(body, out_shapes):
    return pl.pallas_call(
        body,
        out_shape=[jax.ShapeDtypeStruct(s, jnp.float32) for s in out_shapes])


def kernel(x, edge_index, batch,
           W1, a_src1, a_dst1, b1, bn1_g, bn1_b,
           W2, a_src2, a_dst2, b2, bn2_g, bn2_b,
           W3, a_src3, a_dst3, b3, bn3_g, bn3_b,
           W4, a_src4, a_dst4, b4, bn4_g, bn4_b,
           fc1_W, fc1_b, cls_W, cls_b):
    xp = jnp.pad(x, ((0, NPAD - N), (0, 0)))
    loops = jnp.arange(N, dtype=edge_index.dtype)
    srcp = jnp.pad(jnp.concatenate([edge_index[0], loops]), (0, E_PAD - E - N),
                   constant_values=N)
    dstp = jnp.pad(jnp.concatenate([edge_index[1], loops]), (0, E_PAD - E - N),
                   constant_values=N)
    batchp = jnp.pad(batch, (0, NPAD - N), constant_values=G)[:, None]

    col = lambda v: v.reshape(-1, 1)
    row = lambda v: v.reshape(1, -1)

    dims = [32, 64, 128, 64]
    As = [a_src1, a_src2, a_src3, a_src4]
    Ad = [a_dst1, a_dst2, a_dst3, a_dst4]
    Ws = [W1, W2, W3, W4]
    Bs = [b1, b2, b3, b4]
    Gs = [bn1_g, bn2_g, bn3_g, bn4_g]
    Bt = [bn1_b, bn2_b, bn3_b, bn4_b]

    hs, ss, sd = _tc_call(_k0_body, [(2, NPAD, 16), (NPAD, 1), (NPAD, 1)])(
        xp, W1, col(a_src1), col(a_dst1))

    for i in range(4):
        do = dims[i]
        ch = 128
        outp, denp = _sc_edge_pass(do)(
            hs[0], hs[1], ss.reshape(-1), sd.reshape(-1),
            srcp.reshape(E_PAD // ch, ch), dstp.reshape(E_PAD // ch, ch))
        if i < 3:
            dn = dims[i + 1]
            hs, ss, sd = _tc_call(
                _ep_body, [(2, NPAD, dn // 2), (NPAD, 1), (NPAD, 1)])(
                    outp, denp, row(Bs[i]), row(Gs[i]), row(Bt[i]),
                    Ws[i + 1], col(As[i + 1]), col(Ad[i + 1]))
        else:
            (out,) = _tc_call(_final_body, [(G, OUT_DIM)])(
                outp, denp, row(Bs[i]), row(Gs[i]), row(Bt[i]),
                batchp, fc1_W, row(fc1_b), cls_W, row(cls_b))
    return out
